# Initial kernel scaffold; baseline (speedup 1.0000x reference)
import jax
import jax.numpy as jnp
from jax.experimental import pallas as pl


def kernel(x, edge_index, t, init_weight, root_weight, bias):
    raise NotImplementedError("write your pallas kernel here")


# speedup vs baseline: 4.6610x; 4.6610x over previous
import functools

import jax
import jax.numpy as jnp
from jax import lax
from jax.experimental import pallas as pl
from jax.experimental.pallas import tpu as pltpu
from jax.experimental.pallas import tpu_sc as plsc

N = 10000
F = 128
STEPS = 10
TILES = 16
NPAD = 10240
ROWS_PT = NPAD // TILES
RCHUNK = 32
NCHUNKS = ROWS_PT // RCHUNK
ECHUNK = 128
NEWTON_ITERS = 22


def _sc_diffusion_body(x_ref, src_ref, dst_ref, tn_ref, tn0_ref,
                       y_ref, g_ref,
                       acc_sh,
                       gbuf, na, ng, ny, dinv_t, zbuf,
                       idx_s, idx_d, tnv, tn0v, sem):
    wid = lax.axis_index("s")
    r0 = wid * ROWS_PT
    n_echunks = src_ref.shape[1]

    def _zb_row(r, _):
        z16 = jnp.zeros((16,), jnp.float32)
        for j in range(F // 16):
            zbuf[r, pl.ds(j * 16, 16)] = z16
        return 0
    lax.fori_loop(0, RCHUNK, _zb_row, 0)

    def _ones_row(r, _):
        o16 = jnp.ones((16,), jnp.float32)
        for j in range(F // 16):
            gbuf[r, pl.ds(j * 16, 16)] = o16
        return 0
    lax.fori_loop(0, ECHUNK, _ones_row, 0)

    pltpu.sync_copy(tn_ref, tnv)
    pltpu.sync_copy(tn0_ref, tn0v)

    _ch = min(ECHUNK, ROWS_PT)
    for c in range(ROWS_PT // _ch):
        pltpu.sync_copy(gbuf.at[pl.ds(0, _ch)], acc_sh.at[pl.ds(r0 + c * _ch, _ch)])
    plsc.subcore_barrier()

    def _deg(j, _):
        pltpu.sync_copy(dst_ref.at[wid, j], idx_d)
        pltpu.sync_copy(gbuf, acc_sh.at[idx_d], add=True)
        return 0
    lax.fori_loop(0, n_echunks, _deg, 0)
    plsc.subcore_barrier()

    def _dinv_chunk(c, _):
        base = r0 + c * RCHUNK
        pltpu.sync_copy(acc_sh.at[pl.ds(base, RCHUNK)], na)

        def _row(r, _):
            d = na[r, pl.ds(0, 16)]
            yv = 1.0 / d
            for _ in range(NEWTON_ITERS):
                yv = yv * (1.5 - 0.5 * d * yv * yv)
            dinv_t[c * RCHUNK + r, :] = yv
            return 0
        lax.fori_loop(0, RCHUNK, _row, 0)
        return 0
    lax.fori_loop(0, NCHUNKS, _dinv_chunk, 0)

    iota16 = lax.broadcasted_iota(jnp.int32, (16,), 0)
    tn0 = tn0v[...]

    def _ginit(c, _):
        base = r0 + c * RCHUNK
        pltpu.sync_copy(x_ref.at[pl.ds(base, RCHUNK)], na)

        def _row(r, _):
            dv = dinv_t[c * RCHUNK + r, :]
            for j in range(F // 16):
                v = na[r, pl.ds(j * 16, 16)]
                ng[r, pl.ds(j * 16, 16)] = dv * v
                ny[r, pl.ds(j * 16, 16)] = tn0 * v
            return 0
        lax.fori_loop(0, RCHUNK, _row, 0)
        pltpu.sync_copy(ng, g_ref.at[pl.ds(base, RCHUNK)])
        pltpu.sync_copy(ny, y_ref.at[pl.ds(base, RCHUNK)])
        return 0
    lax.fori_loop(0, NCHUNKS, _ginit, 0)
    plsc.subcore_barrier()

    def _step(i, _):
        def _z(c, _):
            pltpu.sync_copy(zbuf, acc_sh.at[pl.ds(r0 + c * RCHUNK, RCHUNK)])
            return 0
        lax.fori_loop(0, NCHUNKS, _z, 0)
        plsc.subcore_barrier()

        def _e(j, _):
            pltpu.sync_copy(src_ref.at[wid, j], idx_s)
            pltpu.sync_copy(dst_ref.at[wid, j], idx_d)
            pltpu.async_copy(g_ref.at[idx_s], gbuf, sem).wait()
            pltpu.sync_copy(gbuf, acc_sh.at[idx_d], add=True)
            return 0
        lax.fori_loop(0, n_echunks, _e, 0)
        plsc.subcore_barrier()

        tni = plsc.load_gather(tnv, [iota16 * 0 + i])

        def _n(c, _):
            base = r0 + c * RCHUNK
            pltpu.sync_copy(acc_sh.at[pl.ds(base, RCHUNK)], na)
            pltpu.sync_copy(g_ref.at[pl.ds(base, RCHUNK)], ng)
            pltpu.sync_copy(y_ref.at[pl.ds(base, RCHUNK)], ny)

            def _row(r, _):
                dv = dinv_t[c * RCHUNK + r, :]
                dv2 = dv * dv
                cy = tni * dv
                for j in range(F // 16):
                    tmp = na[r, pl.ds(j * 16, 16)] + ng[r, pl.ds(j * 16, 16)]
                    ny[r, pl.ds(j * 16, 16)] = ny[r, pl.ds(j * 16, 16)] + cy * tmp
                    ng[r, pl.ds(j * 16, 16)] = dv2 * tmp
                return 0
            lax.fori_loop(0, RCHUNK, _row, 0)
            pltpu.sync_copy(ng, g_ref.at[pl.ds(base, RCHUNK)])
            pltpu.sync_copy(ny, y_ref.at[pl.ds(base, RCHUNK)])
            return 0
        lax.fori_loop(0, NCHUNKS, _n, 0)
        plsc.subcore_barrier()
        return 0
    lax.fori_loop(1, STEPS, _step, 0)


@functools.lru_cache(maxsize=None)
def _make_sc_diffusion(n_echunks):
    return pl.kernel(
        _sc_diffusion_body,
        out_type=(
            jax.ShapeDtypeStruct((NPAD, F), jnp.float32),
            jax.ShapeDtypeStruct((NPAD, F), jnp.float32),
        ),
        mesh=plsc.VectorSubcoreMesh(
            core_axis_name="c", subcore_axis_name="s", num_cores=1),
        compiler_params=pltpu.CompilerParams(
            needs_layout_passes=False, use_tc_tiling_on_sc=False),
        scratch_types=[
            pltpu.VMEM_SHARED((NPAD, F), jnp.float32),
            pltpu.VMEM((ECHUNK, F), jnp.float32),
            pltpu.VMEM((RCHUNK, F), jnp.float32),
            pltpu.VMEM((RCHUNK, F), jnp.float32),
            pltpu.VMEM((RCHUNK, F), jnp.float32),
            pltpu.VMEM((ROWS_PT, 16), jnp.float32),
            pltpu.VMEM((RCHUNK, F), jnp.float32),
            pltpu.VMEM((ECHUNK,), jnp.int32),
            pltpu.VMEM((ECHUNK,), jnp.int32),
            pltpu.VMEM((16,), jnp.float32),
            pltpu.VMEM((16,), jnp.float32),
            pltpu.SemaphoreType.DMA,
        ],
    )


def _tnorm_body(t_ref, o_ref):
    tv = t_ref[...]
    ex = jnp.exp(tv - jnp.max(tv))
    tn = ex / jnp.sum(ex)
    o_ref[0:1, :] = tn
    o_ref[1:2, :] = jnp.full((1, 128), tn[0, 0], jnp.float32)


def _tnorm(t):
    t_p = jnp.full((1, 128), -jnp.inf, jnp.float32).at[0, :t.shape[0]].set(t)
    return pl.pallas_call(
        _tnorm_body,
        out_shape=jax.ShapeDtypeStruct((2, 128), jnp.float32),
    )(t_p)


def _head_body(y_ref, w1_ref, w2_ref, b_ref, o_ref):
    w = w1_ref[0] + w2_ref[0, 0]
    acc = jnp.dot(y_ref[...], w, preferred_element_type=jnp.float32)
    o_ref[...] = jnp.maximum(acc + b_ref[0, 0], 0.0)


def _arma_head(y, init_weight, root_weight, bias):
    blk = 1000
    grid = (N // blk,)
    return pl.pallas_call(
        _head_body,
        grid=grid,
        in_specs=[
            pl.BlockSpec((blk, F), lambda i: (i, 0)),
            pl.BlockSpec(init_weight.shape, lambda i: (0, 0, 0)),
            pl.BlockSpec(root_weight.shape, lambda i: (0, 0, 0, 0)),
            pl.BlockSpec(bias.shape, lambda i: (0, 0, 0, 0)),
        ],
        out_specs=pl.BlockSpec((blk, F), lambda i: (i, 0)),
        out_shape=jax.ShapeDtypeStruct((N, F), jnp.float32),
    )(y, init_weight, root_weight, bias)


def kernel(x, edge_index, t, init_weight, root_weight, bias):
    src = edge_index[0]
    dst = edge_index[1]
    e = src.shape[0]
    ept = -(-e // (TILES * ECHUNK)) * ECHUNK
    pad = TILES * ept - e
    src_p = jnp.concatenate(
        [src, jnp.full((pad,), N, jnp.int32)]).reshape(TILES, ept // ECHUNK, ECHUNK)
    dst_p = jnp.concatenate(
        [dst, jnp.full((pad,), N, jnp.int32)]).reshape(TILES, ept // ECHUNK, ECHUNK)
    x_p = jnp.zeros((NPAD, F), jnp.float32).at[:N].set(x)
    tn_all = _tnorm(t)
    tn = tn_all[0, :16]
    tn0 = tn_all[1, :16]

    y_full, _ = _make_sc_diffusion(ept // ECHUNK)(x_p, src_p, dst_p, tn, tn0)
    return _arma_head(y_full[:N], init_weight, root_weight, bias)


# speedup vs baseline: 4.8527x; 1.0411x over previous
import functools

import jax
import jax.numpy as jnp
from jax import lax
from jax.experimental import pallas as pl
from jax.experimental.pallas import tpu as pltpu
from jax.experimental.pallas import tpu_sc as plsc

N = 10000
F = 128
STEPS = 10
TILES = 16
NPAD = 10240
ROWS_PT = NPAD // TILES
RCHUNK = 32
NCHUNKS = ROWS_PT // RCHUNK
ECHUNK = 128
EBLK = 8
NEWTON_ITERS = 22


def _sc_diffusion_body(x_ref, src_ref, dst_ref, tn_ref, tn0_ref,
                       y_ref, g_ref,
                       acc_sh,
                       gbuf, na, ng, ny, dinv_c,
                       idx_s, idx_d, tnv, tn0v,
                       sg0, sg1, ss0, ss1):
    wid = lax.axis_index("s")
    r0 = wid * ROWS_PT
    n_eblks = src_ref.shape[1] // EBLK
    iota16 = lax.broadcasted_iota(jnp.int32, (16,), 0)

    pltpu.sync_copy(tn_ref, tnv)
    pltpu.sync_copy(tn0_ref, tn0v)

    def _ones_row(r, _):
        o16 = jnp.ones((16,), jnp.float32)
        for j in range(F // 16):
            gbuf[0, r, pl.ds(j * 16, 16)] = o16
        return 0
    lax.fori_loop(0, ECHUNK, _ones_row, 0)

    _ch = min(ECHUNK, ROWS_PT)
    for c in range(ROWS_PT // _ch):
        pltpu.sync_copy(gbuf.at[0, pl.ds(0, _ch)],
                        acc_sh.at[pl.ds(r0 + c * _ch, _ch)])
    plsc.subcore_barrier()

    def _deg(jb, _):
        pltpu.sync_copy(dst_ref.at[wid, pl.ds(jb * EBLK, EBLK)], idx_d)
        for k in range(EBLK):
            pltpu.sync_copy(gbuf.at[0], acc_sh.at[idx_d.at[k]], add=True)
        return 0
    lax.fori_loop(0, n_eblks, _deg, 0)
    plsc.subcore_barrier()

    def _dinv16(k, _):
        pltpu.sync_copy(acc_sh.at[pl.ds(r0 + k * 16, 16)], na.at[pl.ds(0, 16)])
        d = plsc.load_gather(na, [iota16, iota16 * 0 + 1])
        yv = 1.0 / d
        for _i in range(NEWTON_ITERS):
            yv = yv * (1.5 - 0.5 * d * yv * yv)
        dinv_c[pl.ds(k * 16, 16)] = yv
        return 0
    lax.fori_loop(0, ROWS_PT // 16, _dinv16, 0)

    tn0 = tn0v[...]

    def _ginit(c, _):
        base = r0 + c * RCHUNK
        pltpu.sync_copy(x_ref.at[pl.ds(base, RCHUNK)], na)

        def _row(r, _):
            dv = plsc.load_gather(dinv_c, [iota16 * 0 + (c * RCHUNK + r)])
            for j in range(F // 16):
                v = na[r, pl.ds(j * 16, 16)]
                ng[r, pl.ds(j * 16, 16)] = dv * v
                ny[r, pl.ds(j * 16, 16)] = tn0 * v
            return 0
        lax.fori_loop(0, RCHUNK, _row, 0)
        pltpu.sync_copy(ng, g_ref.at[pl.ds(base, RCHUNK)])
        pltpu.sync_copy(ny, y_ref.at[pl.ds(base, RCHUNK)])
        return 0
    lax.fori_loop(0, NCHUNKS, _ginit, 0)
    plsc.subcore_barrier()

    def _step(i, _):
        def _zb(r, _):
            z16 = jnp.zeros((16,), jnp.float32)
            for j in range(F // 16):
                na[r, pl.ds(j * 16, 16)] = z16
            return 0
        lax.fori_loop(0, RCHUNK, _zb, 0)

        def _z(c, _):
            pltpu.sync_copy(na, acc_sh.at[pl.ds(r0 + c * RCHUNK, RCHUNK)])
            return 0
        lax.fori_loop(0, NCHUNKS, _z, 0)
        plsc.subcore_barrier()

        def _eblk(jb, _):
            pltpu.sync_copy(src_ref.at[wid, pl.ds(jb * EBLK, EBLK)], idx_s)
            pltpu.sync_copy(dst_ref.at[wid, pl.ds(jb * EBLK, EBLK)], idx_d)
            sg = (sg0, sg1)
            ss = (ss0, ss1)
            gds = [None, None]
            sds = [None, None]
            gds[0] = pltpu.async_copy(g_ref.at[idx_s.at[0]], gbuf.at[0], sg[0])
            gds[1] = pltpu.async_copy(g_ref.at[idx_s.at[1]], gbuf.at[1], sg[1])
            for k in range(EBLK):
                b = k % 2
                gds[b].wait()
                sds[b] = pltpu.async_copy(
                    gbuf.at[b], acc_sh.at[idx_d.at[k]], ss[b], add=True)
                if k + 2 < EBLK:
                    sds[b].wait()
                    gds[b] = pltpu.async_copy(
                        g_ref.at[idx_s.at[k + 2]], gbuf.at[b], sg[b])
            sds[0].wait()
            sds[1].wait()
            return 0
        lax.fori_loop(0, n_eblks, _eblk, 0)
        plsc.subcore_barrier()

        tni = plsc.load_gather(tnv, [iota16 * 0 + i])

        def _n(c, _):
            base = r0 + c * RCHUNK
            pltpu.sync_copy(acc_sh.at[pl.ds(base, RCHUNK)], na)
            pltpu.sync_copy(g_ref.at[pl.ds(base, RCHUNK)], ng)
            pltpu.sync_copy(y_ref.at[pl.ds(base, RCHUNK)], ny)

            def _row(r, _):
                dv = plsc.load_gather(dinv_c, [iota16 * 0 + (c * RCHUNK + r)])
                dv2 = dv * dv
                cy = tni * dv
                for j in range(F // 16):
                    tmp = na[r, pl.ds(j * 16, 16)] + ng[r, pl.ds(j * 16, 16)]
                    ny[r, pl.ds(j * 16, 16)] = ny[r, pl.ds(j * 16, 16)] + cy * tmp
                    ng[r, pl.ds(j * 16, 16)] = dv2 * tmp
                return 0
            lax.fori_loop(0, RCHUNK, _row, 0)
            pltpu.sync_copy(ng, g_ref.at[pl.ds(base, RCHUNK)])
            pltpu.sync_copy(ny, y_ref.at[pl.ds(base, RCHUNK)])
            return 0
        lax.fori_loop(0, NCHUNKS, _n, 0)
        plsc.subcore_barrier()
        return 0
    lax.fori_loop(1, STEPS, _step, 0)


@functools.lru_cache(maxsize=None)
def _make_sc_diffusion(n_echunks):
    return pl.kernel(
        _sc_diffusion_body,
        out_type=(
            jax.ShapeDtypeStruct((NPAD, F), jnp.float32),
            jax.ShapeDtypeStruct((NPAD, F), jnp.float32),
        ),
        mesh=plsc.VectorSubcoreMesh(
            core_axis_name="c", subcore_axis_name="s", num_cores=1),
        compiler_params=pltpu.CompilerParams(
            needs_layout_passes=False, use_tc_tiling_on_sc=False),
        scratch_types=[
            pltpu.VMEM_SHARED((NPAD, F), jnp.float32),
            pltpu.VMEM((2, ECHUNK, F), jnp.float32),
            pltpu.VMEM((RCHUNK, F), jnp.float32),
            pltpu.VMEM((RCHUNK, F), jnp.float32),
            pltpu.VMEM((RCHUNK, F), jnp.float32),
            pltpu.VMEM((ROWS_PT,), jnp.float32),
            pltpu.VMEM((EBLK, ECHUNK), jnp.int32),
            pltpu.VMEM((EBLK, ECHUNK), jnp.int32),
            pltpu.VMEM((16,), jnp.float32),
            pltpu.VMEM((16,), jnp.float32),
            pltpu.SemaphoreType.DMA,
            pltpu.SemaphoreType.DMA,
            pltpu.SemaphoreType.DMA,
            pltpu.SemaphoreType.DMA,
        ],
    )


def _tnorm_body(t_ref, o_ref):
    tv = t_ref[...]
    ex = jnp.exp(tv - jnp.max(tv))
    tn = ex / jnp.sum(ex)
    o_ref[0:1, :] = tn
    o_ref[1:2, :] = jnp.full((1, 128), tn[0, 0], jnp.float32)


def _tnorm(t):
    t_p = jnp.full((1, 128), -jnp.inf, jnp.float32).at[0, :t.shape[0]].set(t)
    return pl.pallas_call(
        _tnorm_body,
        out_shape=jax.ShapeDtypeStruct((2, 128), jnp.float32),
    )(t_p)


def _head_body(y_ref, w1_ref, w2_ref, b_ref, o_ref):
    w = w1_ref[0] + w2_ref[0, 0]
    acc = jnp.dot(y_ref[...], w, preferred_element_type=jnp.float32)
    o_ref[...] = jnp.maximum(acc + b_ref[0, 0], 0.0)


def _arma_head(y, init_weight, root_weight, bias):
    blk = 1000
    grid = (N // blk,)
    return pl.pallas_call(
        _head_body,
        grid=grid,
        in_specs=[
            pl.BlockSpec((blk, F), lambda i: (i, 0)),
            pl.BlockSpec(init_weight.shape, lambda i: (0, 0, 0)),
            pl.BlockSpec(root_weight.shape, lambda i: (0, 0, 0, 0)),
            pl.BlockSpec(bias.shape, lambda i: (0, 0, 0, 0)),
        ],
        out_specs=pl.BlockSpec((blk, F), lambda i: (i, 0)),
        out_shape=jax.ShapeDtypeStruct((N, F), jnp.float32),
    )(y, init_weight, root_weight, bias)


def kernel(x, edge_index, t, init_weight, root_weight, bias):
    src = edge_index[0]
    dst = edge_index[1]
    e = src.shape[0]
    ept = -(-e // (TILES * ECHUNK * EBLK)) * (ECHUNK * EBLK)
    pad = TILES * ept - e
    src_p = jnp.concatenate(
        [src, jnp.full((pad,), N, jnp.int32)]).reshape(TILES, ept // ECHUNK, ECHUNK)
    dst_p = jnp.concatenate(
        [dst, jnp.full((pad,), N, jnp.int32)]).reshape(TILES, ept // ECHUNK, ECHUNK)
    x_p = jnp.zeros((NPAD, F), jnp.float32).at[:N].set(x)
    tn_all = _tnorm(t)
    tn = tn_all[0, :16]
    tn0 = tn_all[1, :16]

    y_full, _ = _make_sc_diffusion(ept // ECHUNK)(x_p, src_p, dst_p, tn, tn0)
    return _arma_head(y_full[:N], init_weight, root_weight, bias)


# speedup vs baseline: 4.9871x; 1.0277x over previous
import functools

import jax
import jax.numpy as jnp
from jax import lax
from jax.experimental import pallas as pl
from jax.experimental.pallas import tpu as pltpu
from jax.experimental.pallas import tpu_sc as plsc

N = 10000
F = 128
STEPS = 10
TILES = 16
NPAD = 10240
ROWS_PT = NPAD // TILES
RCHUNK = 32
NCHUNKS = ROWS_PT // RCHUNK
ECHUNK = 64
EBLK = 16
NBUF = 4
NEWTON_ITERS = 22


def _sc_diffusion_body(x_ref, src_ref, dst_ref, tn_ref, tn0_ref,
                       y_ref, g_ref,
                       acc_sh,
                       gbuf, na, ng, ny, dinv_c,
                       idx_s, idx_d, tnv, tn0v,
                       sg0, sg1, sg2, sg3, ss0, ss1, ss2, ss3):
    wid = lax.axis_index("s")
    r0 = wid * ROWS_PT
    n_eblks = src_ref.shape[1] // EBLK
    iota16 = lax.broadcasted_iota(jnp.int32, (16,), 0)

    pltpu.sync_copy(tn_ref, tnv)
    pltpu.sync_copy(tn0_ref, tn0v)

    def _ones_row(r, _):
        o16 = jnp.ones((16,), jnp.float32)
        for j in range(F // 16):
            gbuf[0, r, pl.ds(j * 16, 16)] = o16
        return 0
    lax.fori_loop(0, ECHUNK, _ones_row, 0)

    _ch = min(ECHUNK, ROWS_PT)
    for c in range(ROWS_PT // _ch):
        pltpu.sync_copy(gbuf.at[0, pl.ds(0, _ch)],
                        acc_sh.at[pl.ds(r0 + c * _ch, _ch)])
    plsc.subcore_barrier()

    def _deg(jb, _):
        pltpu.sync_copy(dst_ref.at[wid, pl.ds(jb * EBLK, EBLK)], idx_d)
        for k in range(EBLK):
            pltpu.sync_copy(gbuf.at[0], acc_sh.at[idx_d.at[k]], add=True)
        return 0
    lax.fori_loop(0, n_eblks, _deg, 0)
    plsc.subcore_barrier()

    def _dinv16(k, _):
        pltpu.sync_copy(acc_sh.at[pl.ds(r0 + k * 16, 16)], na.at[pl.ds(0, 16)])
        d = plsc.load_gather(na, [iota16, iota16 * 0 + 1])
        yv = 1.0 / d
        for _i in range(NEWTON_ITERS):
            yv = yv * (1.5 - 0.5 * d * yv * yv)
        dinv_c[pl.ds(k * 16, 16)] = yv
        return 0
    lax.fori_loop(0, ROWS_PT // 16, _dinv16, 0)

    tn0 = tn0v[...]

    def _ginit(c, _):
        base = r0 + c * RCHUNK
        pltpu.sync_copy(x_ref.at[pl.ds(base, RCHUNK)], na)

        def _row(r, _):
            dv = plsc.load_gather(dinv_c, [iota16 * 0 + (c * RCHUNK + r)])
            for j in range(F // 16):
                v = na[r, pl.ds(j * 16, 16)]
                ng[r, pl.ds(j * 16, 16)] = dv * v
                ny[r, pl.ds(j * 16, 16)] = tn0 * v
            return 0
        lax.fori_loop(0, RCHUNK, _row, 0)
        pltpu.sync_copy(ng, g_ref.at[pl.ds(base, RCHUNK)])
        pltpu.sync_copy(ny, y_ref.at[pl.ds(base, RCHUNK)])
        return 0
    lax.fori_loop(0, NCHUNKS, _ginit, 0)
    plsc.subcore_barrier()

    def _step(i, _):
        def _zb(r, _):
            z16 = jnp.zeros((16,), jnp.float32)
            for j in range(F // 16):
                na[r, pl.ds(j * 16, 16)] = z16
            return 0
        lax.fori_loop(0, RCHUNK, _zb, 0)

        def _z(c, _):
            pltpu.sync_copy(na, acc_sh.at[pl.ds(r0 + c * RCHUNK, RCHUNK)])
            return 0
        lax.fori_loop(0, NCHUNKS, _z, 0)
        plsc.subcore_barrier()

        def _eblk(jb, _):
            pltpu.sync_copy(src_ref.at[wid, pl.ds(jb * EBLK, EBLK)], idx_s)
            pltpu.sync_copy(dst_ref.at[wid, pl.ds(jb * EBLK, EBLK)], idx_d)
            sg = (sg0, sg1, sg2, sg3)
            ss = (ss0, ss1, ss2, ss3)
            gds = [None] * NBUF
            sds = [None] * NBUF
            for b in range(NBUF):
                gds[b] = pltpu.async_copy(
                    g_ref.at[idx_s.at[b]], gbuf.at[b], sg[b])
            for k in range(EBLK):
                b = k % NBUF
                gds[b].wait()
                sds[b] = pltpu.async_copy(
                    gbuf.at[b], acc_sh.at[idx_d.at[k]], ss[b], add=True)
                if k + NBUF < EBLK:
                    sds[b].wait()
                    gds[b] = pltpu.async_copy(
                        g_ref.at[idx_s.at[k + NBUF]], gbuf.at[b], sg[b])
            for b in range(NBUF):
                sds[b].wait()
            return 0
        lax.fori_loop(0, n_eblks, _eblk, 0)
        plsc.subcore_barrier()

        tni = plsc.load_gather(tnv, [iota16 * 0 + i])

        def _n(c, _):
            base = r0 + c * RCHUNK
            pltpu.sync_copy(acc_sh.at[pl.ds(base, RCHUNK)], na)
            pltpu.sync_copy(g_ref.at[pl.ds(base, RCHUNK)], ng)
            pltpu.sync_copy(y_ref.at[pl.ds(base, RCHUNK)], ny)

            def _row(r, _):
                dv = plsc.load_gather(dinv_c, [iota16 * 0 + (c * RCHUNK + r)])
                dv2 = dv * dv
                cy = tni * dv
                for j in range(F // 16):
                    tmp = na[r, pl.ds(j * 16, 16)] + ng[r, pl.ds(j * 16, 16)]
                    ny[r, pl.ds(j * 16, 16)] = ny[r, pl.ds(j * 16, 16)] + cy * tmp
                    ng[r, pl.ds(j * 16, 16)] = dv2 * tmp
                return 0
            lax.fori_loop(0, RCHUNK, _row, 0)
            pltpu.sync_copy(ng, g_ref.at[pl.ds(base, RCHUNK)])
            pltpu.sync_copy(ny, y_ref.at[pl.ds(base, RCHUNK)])
            return 0
        lax.fori_loop(0, NCHUNKS, _n, 0)
        plsc.subcore_barrier()
        return 0
    lax.fori_loop(1, STEPS, _step, 0)


@functools.lru_cache(maxsize=None)
def _make_sc_diffusion(n_echunks):
    return pl.kernel(
        _sc_diffusion_body,
        out_type=(
            jax.ShapeDtypeStruct((NPAD, F), jnp.float32),
            jax.ShapeDtypeStruct((NPAD, F), jnp.float32),
        ),
        mesh=plsc.VectorSubcoreMesh(
            core_axis_name="c", subcore_axis_name="s", num_cores=1),
        compiler_params=pltpu.CompilerParams(
            needs_layout_passes=False, use_tc_tiling_on_sc=False),
        scratch_types=[
            pltpu.VMEM_SHARED((NPAD, F), jnp.float32),
            pltpu.VMEM((NBUF, ECHUNK, F), jnp.float32),
            pltpu.VMEM((RCHUNK, F), jnp.float32),
            pltpu.VMEM((RCHUNK, F), jnp.float32),
            pltpu.VMEM((RCHUNK, F), jnp.float32),
            pltpu.VMEM((ROWS_PT,), jnp.float32),
            pltpu.VMEM((EBLK, ECHUNK), jnp.int32),
            pltpu.VMEM((EBLK, ECHUNK), jnp.int32),
            pltpu.VMEM((16,), jnp.float32),
            pltpu.VMEM((16,), jnp.float32),
        ] + [pltpu.SemaphoreType.DMA] * 8,
    )


def _tnorm_body(t_ref, o_ref):
    tv = t_ref[...]
    ex = jnp.exp(tv - jnp.max(tv))
    tn = ex / jnp.sum(ex)
    o_ref[0:1, :] = tn
    o_ref[1:2, :] = jnp.full((1, 128), tn[0, 0], jnp.float32)


def _tnorm(t):
    t_p = jnp.full((1, 128), -jnp.inf, jnp.float32).at[0, :t.shape[0]].set(t)
    return pl.pallas_call(
        _tnorm_body,
        out_shape=jax.ShapeDtypeStruct((2, 128), jnp.float32),
    )(t_p)


def _head_body(y_ref, w1_ref, w2_ref, b_ref, o_ref):
    w = w1_ref[0] + w2_ref[0, 0]
    acc = jnp.dot(y_ref[...], w, preferred_element_type=jnp.float32)
    o_ref[...] = jnp.maximum(acc + b_ref[0, 0], 0.0)


def _arma_head(y, init_weight, root_weight, bias):
    blk = 1000
    grid = (N // blk,)
    return pl.pallas_call(
        _head_body,
        grid=grid,
        in_specs=[
            pl.BlockSpec((blk, F), lambda i: (i, 0)),
            pl.BlockSpec(init_weight.shape, lambda i: (0, 0, 0)),
            pl.BlockSpec(root_weight.shape, lambda i: (0, 0, 0, 0)),
            pl.BlockSpec(bias.shape, lambda i: (0, 0, 0, 0)),
        ],
        out_specs=pl.BlockSpec((blk, F), lambda i: (i, 0)),
        out_shape=jax.ShapeDtypeStruct((N, F), jnp.float32),
    )(y, init_weight, root_weight, bias)


def kernel(x, edge_index, t, init_weight, root_weight, bias):
    src = edge_index[0]
    dst = edge_index[1]
    e = src.shape[0]
    ept = -(-e // (TILES * ECHUNK * EBLK)) * (ECHUNK * EBLK)
    pad = TILES * ept - e
    src_p = jnp.concatenate(
        [src, jnp.full((pad,), N, jnp.int32)]).reshape(TILES, ept // ECHUNK, ECHUNK)
    dst_p = jnp.concatenate(
        [dst, jnp.full((pad,), N, jnp.int32)]).reshape(TILES, ept // ECHUNK, ECHUNK)
    x_p = jnp.zeros((NPAD, F), jnp.float32).at[:N].set(x)
    tn_all = _tnorm(t)
    tn = tn_all[0, :16]
    tn0 = tn_all[1, :16]

    y_full, _ = _make_sc_diffusion(ept // ECHUNK)(x_p, src_p, dst_p, tn, tn0)
    return _arma_head(y_full[:N], init_weight, root_weight, bias)
